# baseline (speedup 1.0000x reference)
import functools

import jax
import jax.numpy as jnp
from jax import lax
from jax.experimental import pallas as pl
from jax.experimental.pallas import tpu as pltpu
from jax.experimental.pallas import tpu_sc as plsc

DIM = 192
NUM_HEADS = 6
HEAD_DIM = DIM // NUM_HEADS
TOPK = 32
EPS = 1e-6
HW = 224 * 224
K_SP = 196

_BN = 3584


def _lnqkv_body(x_ref, aff_ref, w_ref, g_ref, b_ref,
                q_ref, k_ref, v_ref, lab_ref):
    x = x_ref[...]
    mu = jnp.mean(x, axis=0, keepdims=True)
    var = jnp.mean((x - mu) * (x - mu), axis=0, keepdims=True)
    xn = (x - mu) * jax.lax.rsqrt(var + EPS)
    xn = xn * g_ref[...] + b_ref[...]
    xnt = jnp.transpose(xn)
    w = w_ref[...]
    out = jax.lax.dot_general(
        xnt, w, (((1,), (1,)), ((), ())),
        preferred_element_type=jnp.float32,
    )
    q_ref[...] = out[:, 0:DIM]
    k_ref[...] = out[:, DIM:2 * DIM]
    v_ref[...] = out[:, 2 * DIM:3 * DIM]
    a = aff_ref[...]
    cmax = jnp.max(a, axis=0, keepdims=True)
    rows = jax.lax.broadcasted_iota(jnp.int32, a.shape, 0)
    lab = jnp.min(jnp.where(a == cmax, rows, K_SP), axis=0)
    lab_ref[...] = lab[None, :]


def _lnqkv(xf, aff, w_all, gamma, beta):
    n = xf.shape[-1]
    grid = n // _BN
    out_shape = [
        jax.ShapeDtypeStruct((n, DIM), jnp.float32),
        jax.ShapeDtypeStruct((n, DIM), jnp.float32),
        jax.ShapeDtypeStruct((n, DIM), jnp.float32),
        jax.ShapeDtypeStruct((1, n), jnp.int32),
    ]
    return pl.pallas_call(
        _lnqkv_body,
        grid=(grid,),
        in_specs=[
            pl.BlockSpec((DIM, _BN), lambda i: (0, i)),
            pl.BlockSpec((K_SP, _BN), lambda i: (0, i)),
            pl.BlockSpec((3 * DIM, DIM), lambda i: (0, 0)),
            pl.BlockSpec((DIM, 1), lambda i: (0, 0)),
            pl.BlockSpec((DIM, 1), lambda i: (0, 0)),
        ],
        out_specs=[
            pl.BlockSpec((_BN, DIM), lambda i: (i, 0)),
            pl.BlockSpec((_BN, DIM), lambda i: (i, 0)),
            pl.BlockSpec((_BN, DIM), lambda i: (i, 0)),
            pl.BlockSpec((1, _BN), lambda i: (0, i)),
        ],
        out_shape=out_shape,
    )(xf, aff, w_all, gamma.reshape(DIM, 1), beta.reshape(DIM, 1))


def _transpose_body(i_ref, o_ref):
    o_ref[...] = jnp.transpose(i_ref[...])


def _to_chw(rows):
    n = rows.shape[0]
    grid = n // _BN
    return pl.pallas_call(
        _transpose_body,
        grid=(grid,),
        in_specs=[pl.BlockSpec((_BN, DIM), lambda i: (i, 0))],
        out_specs=pl.BlockSpec((DIM, _BN), lambda i: (0, i)),
        out_shape=jax.ShapeDtypeStruct((DIM, n), jnp.float32),
    )(rows)


_SPB = 7


def _attn_body(q_ref, k_ref, v_ref, sim_ref, lab_ref, o_ref):
    i = pl.program_id(0)
    rr = jax.lax.broadcasted_iota(jnp.int32, (DIM, DIM), 0) // HEAD_DIM
    cc = jax.lax.broadcasted_iota(jnp.int32, (DIM, DIM), 1) // HEAD_DIM
    bm = (rr == cc).astype(jnp.float32)
    scale = HEAD_DIM ** (-0.5)

    for j in range(_SPB):
        sp = i * _SPB + j
        lab = lab_ref[:, j, :]
        w = sim_ref[:, j, :]
        mf = (lab == sp).astype(jnp.float32)
        mcol = jnp.transpose(mf)
        wcol = jnp.transpose(w)
        q = q_ref[pl.ds(j * TOPK, TOPK), :]
        k = k_ref[pl.ds(j * TOPK, TOPK), :] * mcol
        v = v_ref[pl.ds(j * TOPK, TOPK), :] * (mcol * wcol)
        km = jnp.concatenate([k] * NUM_HEADS, axis=0) * bm
        vm = jnp.concatenate([v] * NUM_HEADS, axis=0) * bm
        s2 = jax.lax.dot_general(
            q, km, (((1,), (1,)), ((), ())),
            preferred_element_type=jnp.float32)
        e = jnp.exp(s2 * scale)
        denom = jax.lax.dot_general(
            e, bm, (((1,), (0,)), ((), ())),
            preferred_element_type=jnp.float32)
        p = e / denom
        out = jax.lax.dot_general(
            p, vm, (((1,), (0,)), ((), ())),
            preferred_element_type=jnp.float32)
        o_ref[pl.ds(j * TOPK, TOPK), :] = out * (mcol * wcol)


def _attention(qg, kg, vg, sims, lab):
    rows = K_SP * TOPK
    grid = K_SP // _SPB
    blk = _SPB * TOPK
    return pl.pallas_call(
        _attn_body,
        grid=(grid,),
        in_specs=[
            pl.BlockSpec((blk, DIM), lambda i: (i, 0)),
            pl.BlockSpec((blk, DIM), lambda i: (i, 0)),
            pl.BlockSpec((blk, DIM), lambda i: (i, 0)),
            pl.BlockSpec((1, _SPB, TOPK), lambda i: (i, 0, 0)),
            pl.BlockSpec((1, _SPB, TOPK), lambda i: (i, 0, 0)),
        ],
        out_specs=pl.BlockSpec((blk, DIM), lambda i: (i, 0)),
        out_shape=jax.ShapeDtypeStruct((rows, DIM), jnp.float32),
    )(qg, kg, vg,
      sims.reshape(grid, _SPB, TOPK),
      lab.reshape(grid, _SPB, TOPK))


_NW = 32
_WIN = 3584
_NWIN = HW // _WIN
_VPW = _WIN // 16
_NEG = -3.4e38


def _topk_rows(aff, sims_out, idx_out, buf0, buf1, stage_f, stage_i, sem0, sem1):
    cid = lax.axis_index("c")
    sid = lax.axis_index("s")
    wid = sid * 2 + cid
    lane = lax.iota(jnp.int32, 16)

    bufs = (buf0, buf1)
    sems = (sem0, sem1)

    def consume_window(buf, wstart, state):
        def insert_vreg(x, base, st):
            tv0, tv1, ti0, ti1, t = st
            m = x > t
            cnt = plsc.all_reduce_population_count(m)[0]

            def wbody(_, c):
                m, tv0, tv1, ti0, ti1, t = c
                jv = plsc.all_reduce_ffs(m)
                xc = lax.gather(
                    x, jv[:, None],
                    lax.GatherDimensionNumbers(
                        offset_dims=(), collapsed_slice_dims=(0,),
                        start_index_map=(0,)),
                    (1,), mode=lax.GatherScatterMode.PROMISE_IN_BOUNDS)
                iv = jv + base
                elig = xc > t
                is0 = tv0 == t
                is1 = tv1 == t
                c0 = jnp.where(is0, ti0, -1)
                c1 = jnp.where(is1, ti1, -1)
                ev = jnp.maximum(jnp.max(c0), jnp.max(c1))
                evv = jnp.full((16,), ev, jnp.int32)
                s0 = is0 & (ti0 == evv) & elig
                s1 = is1 & (ti1 == evv) & elig
                tv0n = jnp.where(s0, xc, tv0)
                tv1n = jnp.where(s1, xc, tv1)
                ti0n = jnp.where(s0, iv, ti0)
                ti1n = jnp.where(s1, iv, ti1)
                tn = jnp.minimum(jnp.min(tv0n), jnp.min(tv1n))
                tnv = jnp.full((16,), tn, jnp.float32)
                mn = m & (lane != jv)
                return (mn, tv0n, tv1n, ti0n, ti1n, tnv)

            fin = lax.fori_loop(0, cnt, wbody, (m, tv0, tv1, ti0, ti1, t))
            return fin[1:]

        def blk_body(bi, st):
            xs = [buf[pl.ds(bi * 256 + r * 16, 16)] for r in range(16)]
            mx = xs[0]
            for r in range(1, 16):
                mx = jnp.maximum(mx, xs[r])
            hit = plsc.all_reduce_population_count(mx > st[4])[0] > 0

            def dirty(st):
                for r in range(16):
                    st = insert_vreg(xs[r], wstart + bi * 256 + r * 16, st)
                return st

            return lax.cond(hit, dirty, lambda s: s, st)

        return lax.fori_loop(0, _VPW // 16, blk_body, state)

    def process_row(row):
        for b in range(2):
            pltpu.make_async_copy(aff.at[row, pl.ds(b * _WIN, _WIN)],
                                  bufs[b], sems[b]).start()

        init = (jnp.full((16,), _NEG, jnp.float32),
                jnp.full((16,), _NEG, jnp.float32),
                lane, lane + 16,
                jnp.full((16,), _NEG, jnp.float32))

        def pair_body(p, state):
            w0 = p * 2
            for b in range(2):
                w = w0 + b
                pltpu.make_async_copy(aff.at[row, pl.ds(w * _WIN, _WIN)],
                                      bufs[b], sems[b]).wait()
                state = consume_window(bufs[b], w * _WIN, state)

                @pl.when(w + 2 < _NWIN)
                def _():
                    pltpu.make_async_copy(
                        aff.at[row, pl.ds((w + 2) * _WIN, _WIN)],
                        bufs[b], sems[b]).start()
            return state

        tv0, tv1, ti0, ti1, t = lax.fori_loop(0, _NWIN // 2, pair_body, init)
        stage_f[pl.ds(0, 16)] = tv0
        stage_f[pl.ds(16, 16)] = tv1
        stage_i[pl.ds(0, 16)] = ti0
        stage_i[pl.ds(16, 16)] = ti1
        pltpu.sync_copy(stage_f, sims_out.at[row])
        pltpu.sync_copy(stage_i, idx_out.at[row])

    def row_body(r, carry):
        row = r * _NW + wid

        @pl.when(row < K_SP)
        def _():
            process_row(row)

        return carry

    lax.fori_loop(0, 7, row_body, 0)


def _sc_topk(aff):
    mesh = plsc.VectorSubcoreMesh(core_axis_name="c", subcore_axis_name="s")
    f = pl.kernel(
        _topk_rows,
        out_type=[
            jax.ShapeDtypeStruct((K_SP, TOPK), jnp.float32),
            jax.ShapeDtypeStruct((K_SP, TOPK), jnp.int32),
        ],
        mesh=mesh,
        compiler_params=pltpu.CompilerParams(needs_layout_passes=False),
        scratch_types=[
            pltpu.VMEM((_WIN,), jnp.float32),
            pltpu.VMEM((_WIN,), jnp.float32),
            pltpu.VMEM((TOPK,), jnp.float32),
            pltpu.VMEM((TOPK,), jnp.int32),
            pltpu.SemaphoreType.DMA,
            pltpu.SemaphoreType.DMA,
        ],
    )
    return f(aff)


def kernel(x, affinity_matrix, gamma, beta, Wq, Wk, Wv, num_spixels):
    B, C, H, W = x.shape
    hw = H * W
    xf = x.reshape(C, hw)
    aff = affinity_matrix[0]
    w_all = jnp.concatenate([Wq, Wk, Wv], axis=0)

    sims, indices = _sc_topk(aff)
    q_r, k_r, v_r, labels = _lnqkv(xf, aff, w_all, gamma, beta)

    flat_idx = indices.reshape(-1)
    lab = labels[0][flat_idx].reshape(K_SP, TOPK)

    qg = q_r[flat_idx]
    kg = k_r[flat_idx]
    vg = v_r[flat_idx]
    out = _attention(qg, kg, vg, sims, lab)

    new_r = v_r.at[flat_idx, :].add(out)
    return _to_chw(new_r).reshape(B, C, H, W)


# speedup vs baseline: 1.1471x; 1.1471x over previous
import functools

import jax
import jax.numpy as jnp
from jax import lax
from jax.experimental import pallas as pl
from jax.experimental.pallas import tpu as pltpu
from jax.experimental.pallas import tpu_sc as plsc

DIM = 192
NUM_HEADS = 6
HEAD_DIM = DIM // NUM_HEADS
TOPK = 32
EPS = 1e-6
HW = 224 * 224
K_SP = 196

_BN = 3584


def _lnqkv_body(x_ref, aff_ref, w_ref, g_ref, b_ref,
                q_ref, k_ref, v_ref, lab_ref):
    x = x_ref[...]
    mu = jnp.mean(x, axis=0, keepdims=True)
    var = jnp.mean((x - mu) * (x - mu), axis=0, keepdims=True)
    xn = (x - mu) * jax.lax.rsqrt(var + EPS)
    xn = xn * g_ref[...] + b_ref[...]
    xnt = jnp.transpose(xn)
    w = w_ref[...]
    out = jax.lax.dot_general(
        xnt, w, (((1,), (1,)), ((), ())),
        preferred_element_type=jnp.float32,
    )
    q_ref[...] = out[:, 0:DIM]
    k_ref[...] = out[:, DIM:2 * DIM]
    v_ref[...] = out[:, 2 * DIM:3 * DIM]
    a = aff_ref[...]
    cmax = jnp.max(a, axis=0, keepdims=True)
    rows = jax.lax.broadcasted_iota(jnp.int32, a.shape, 0)
    lab = jnp.min(jnp.where(a == cmax, rows, K_SP), axis=0)
    lab_ref[...] = lab[None, :]


def _lnqkv(xf, aff, w_all, gamma, beta):
    n = xf.shape[-1]
    grid = n // _BN
    out_shape = [
        jax.ShapeDtypeStruct((n, DIM), jnp.float32),
        jax.ShapeDtypeStruct((n, DIM), jnp.float32),
        jax.ShapeDtypeStruct((n, DIM), jnp.float32),
        jax.ShapeDtypeStruct((1, n), jnp.int32),
    ]
    return pl.pallas_call(
        _lnqkv_body,
        grid=(grid,),
        in_specs=[
            pl.BlockSpec((DIM, _BN), lambda i: (0, i)),
            pl.BlockSpec((K_SP, _BN), lambda i: (0, i)),
            pl.BlockSpec((3 * DIM, DIM), lambda i: (0, 0)),
            pl.BlockSpec((DIM, 1), lambda i: (0, 0)),
            pl.BlockSpec((DIM, 1), lambda i: (0, 0)),
        ],
        out_specs=[
            pl.BlockSpec((_BN, DIM), lambda i: (i, 0)),
            pl.BlockSpec((_BN, DIM), lambda i: (i, 0)),
            pl.BlockSpec((_BN, DIM), lambda i: (i, 0)),
            pl.BlockSpec((1, _BN), lambda i: (0, i)),
        ],
        out_shape=out_shape,
    )(xf, aff, w_all, gamma.reshape(DIM, 1), beta.reshape(DIM, 1))


def _transpose_body(i_ref, o_ref):
    o_ref[...] = jnp.transpose(i_ref[...])


def _to_chw(rows):
    n = rows.shape[0]
    grid = n // _BN
    return pl.pallas_call(
        _transpose_body,
        grid=(grid,),
        in_specs=[pl.BlockSpec((_BN, DIM), lambda i: (i, 0))],
        out_specs=pl.BlockSpec((DIM, _BN), lambda i: (0, i)),
        out_shape=jax.ShapeDtypeStruct((DIM, n), jnp.float32),
    )(rows)


_SPB = 7


def _attn_body(q_ref, k_ref, v_ref, sim_ref, lab_ref, o_ref):
    i = pl.program_id(0)
    rr = jax.lax.broadcasted_iota(jnp.int32, (DIM, DIM), 0) // HEAD_DIM
    cc = jax.lax.broadcasted_iota(jnp.int32, (DIM, DIM), 1) // HEAD_DIM
    bm = (rr == cc).astype(jnp.float32)
    scale = HEAD_DIM ** (-0.5)

    for j in range(_SPB):
        sp = i * _SPB + j
        lab = lab_ref[:, j, :]
        w = sim_ref[:, j, :]
        mf = (lab == sp).astype(jnp.float32)
        mcol = jnp.transpose(mf)
        wcol = jnp.transpose(w)
        q = q_ref[pl.ds(j * TOPK, TOPK), :]
        k = k_ref[pl.ds(j * TOPK, TOPK), :] * mcol
        v = v_ref[pl.ds(j * TOPK, TOPK), :] * (mcol * wcol)
        km = jnp.concatenate([k] * NUM_HEADS, axis=0) * bm
        vm = jnp.concatenate([v] * NUM_HEADS, axis=0) * bm
        s2 = jax.lax.dot_general(
            q, km, (((1,), (1,)), ((), ())),
            preferred_element_type=jnp.float32)
        e = jnp.exp(s2 * scale)
        denom = jax.lax.dot_general(
            e, bm, (((1,), (0,)), ((), ())),
            preferred_element_type=jnp.float32)
        p = e / denom
        out = jax.lax.dot_general(
            p, vm, (((1,), (0,)), ((), ())),
            preferred_element_type=jnp.float32)
        o_ref[pl.ds(j * TOPK, TOPK), :] = out * (mcol * wcol)


def _attention(qg, kg, vg, sims, lab):
    rows = K_SP * TOPK
    grid = K_SP // _SPB
    blk = _SPB * TOPK
    return pl.pallas_call(
        _attn_body,
        grid=(grid,),
        in_specs=[
            pl.BlockSpec((blk, DIM), lambda i: (i, 0)),
            pl.BlockSpec((blk, DIM), lambda i: (i, 0)),
            pl.BlockSpec((blk, DIM), lambda i: (i, 0)),
            pl.BlockSpec((1, _SPB, TOPK), lambda i: (i, 0, 0)),
            pl.BlockSpec((1, _SPB, TOPK), lambda i: (i, 0, 0)),
        ],
        out_specs=pl.BlockSpec((blk, DIM), lambda i: (i, 0)),
        out_shape=jax.ShapeDtypeStruct((rows, DIM), jnp.float32),
    )(qg, kg, vg,
      sims.reshape(grid, _SPB, TOPK),
      lab.reshape(grid, _SPB, TOPK))


_NW = 32
_WIN = 3584
_NWIN = HW // _WIN
_VPW = _WIN // 16
_NEG = -3.4e38


def _topk_rows(aff, sims_out, idx_out, buf0, buf1, stage_f, stage_i, sem0, sem1):
    cid = lax.axis_index("c")
    sid = lax.axis_index("s")
    wid = sid * 2 + cid
    lane = lax.iota(jnp.int32, 16)

    bufs = (buf0, buf1)
    sems = (sem0, sem1)

    def consume_window(buf, wstart, state):
        def insert_vreg(x, base, st):
            tv0, tv1, ti0, ti1, t = st
            m = x > t
            cnt = plsc.all_reduce_population_count(m)[0]

            def wbody(_, c):
                m, tv0, tv1, ti0, ti1, t = c
                jv = plsc.all_reduce_ffs(m)
                xc = lax.gather(
                    x, jv[:, None],
                    lax.GatherDimensionNumbers(
                        offset_dims=(), collapsed_slice_dims=(0,),
                        start_index_map=(0,)),
                    (1,), mode=lax.GatherScatterMode.PROMISE_IN_BOUNDS)
                iv = jv + base
                elig = xc > t
                is0 = tv0 == t
                is1 = tv1 == t
                c0 = jnp.where(is0, ti0, -1)
                c1 = jnp.where(is1, ti1, -1)
                ev = jnp.maximum(jnp.max(c0), jnp.max(c1))
                evv = jnp.full((16,), ev, jnp.int32)
                s0 = is0 & (ti0 == evv) & elig
                s1 = is1 & (ti1 == evv) & elig
                tv0n = jnp.where(s0, xc, tv0)
                tv1n = jnp.where(s1, xc, tv1)
                ti0n = jnp.where(s0, iv, ti0)
                ti1n = jnp.where(s1, iv, ti1)
                tn = jnp.minimum(jnp.min(tv0n), jnp.min(tv1n))
                tnv = jnp.full((16,), tn, jnp.float32)
                mn = m & (lane != jv)
                return (mn, tv0n, tv1n, ti0n, ti1n, tnv)

            fin = lax.fori_loop(0, cnt, wbody, (m, tv0, tv1, ti0, ti1, t))
            return fin[1:]

        def blk_body(bi, st):
            xs = [buf[pl.ds(bi * 128 + r * 16, 16)] for r in range(8)]
            mx = xs[0]
            for r in range(1, 8):
                mx = jnp.maximum(mx, xs[r])
            hit = plsc.all_reduce_population_count(mx > st[4])[0] > 0

            def dirty(st):
                for r in range(8):
                    st = insert_vreg(xs[r], wstart + bi * 128 + r * 16, st)
                return st

            return lax.cond(hit, dirty, lambda s: s, st)

        return lax.fori_loop(0, _VPW // 8, blk_body, state)

    def process_row(row):
        for b in range(2):
            pltpu.make_async_copy(aff.at[row, pl.ds(b * _WIN, _WIN)],
                                  bufs[b], sems[b]).start()

        init = (jnp.full((16,), _NEG, jnp.float32),
                jnp.full((16,), _NEG, jnp.float32),
                lane, lane + 16,
                jnp.full((16,), _NEG, jnp.float32))

        def pair_body(p, state):
            w0 = p * 2
            for b in range(2):
                w = w0 + b
                pltpu.make_async_copy(aff.at[row, pl.ds(w * _WIN, _WIN)],
                                      bufs[b], sems[b]).wait()
                state = consume_window(bufs[b], w * _WIN, state)

                @pl.when(w + 2 < _NWIN)
                def _():
                    pltpu.make_async_copy(
                        aff.at[row, pl.ds((w + 2) * _WIN, _WIN)],
                        bufs[b], sems[b]).start()
            return state

        tv0, tv1, ti0, ti1, t = lax.fori_loop(0, _NWIN // 2, pair_body, init)
        stage_f[pl.ds(0, 16)] = tv0
        stage_f[pl.ds(16, 16)] = tv1
        stage_i[pl.ds(0, 16)] = ti0
        stage_i[pl.ds(16, 16)] = ti1
        pltpu.sync_copy(stage_f, sims_out.at[row])
        pltpu.sync_copy(stage_i, idx_out.at[row])

    def row_body(r, carry):
        row = r * _NW + wid

        @pl.when(row < K_SP)
        def _():
            process_row(row)

        return carry

    lax.fori_loop(0, 7, row_body, 0)


def _sc_topk(aff):
    mesh = plsc.VectorSubcoreMesh(core_axis_name="c", subcore_axis_name="s")
    f = pl.kernel(
        _topk_rows,
        out_type=[
            jax.ShapeDtypeStruct((K_SP, TOPK), jnp.float32),
            jax.ShapeDtypeStruct((K_SP, TOPK), jnp.int32),
        ],
        mesh=mesh,
        compiler_params=pltpu.CompilerParams(needs_layout_passes=False),
        scratch_types=[
            pltpu.VMEM((_WIN,), jnp.float32),
            pltpu.VMEM((_WIN,), jnp.float32),
            pltpu.VMEM((TOPK,), jnp.float32),
            pltpu.VMEM((TOPK,), jnp.int32),
            pltpu.SemaphoreType.DMA,
            pltpu.SemaphoreType.DMA,
        ],
    )
    return f(aff)


def kernel(x, affinity_matrix, gamma, beta, Wq, Wk, Wv, num_spixels):
    B, C, H, W = x.shape
    hw = H * W
    xf = x.reshape(C, hw)
    aff = affinity_matrix[0]
    w_all = jnp.concatenate([Wq, Wk, Wv], axis=0)

    sims, indices = _sc_topk(aff)
    q_r, k_r, v_r, labels = _lnqkv(xf, aff, w_all, gamma, beta)

    flat_idx = indices.reshape(-1)
    lab = labels[0][flat_idx].reshape(K_SP, TOPK)

    qg = q_r[flat_idx]
    kg = k_r[flat_idx]
    vg = v_r[flat_idx]
    out = _attention(qg, kg, vg, sims, lab)

    new_r = v_r.at[flat_idx, :].add(out)
    return _to_chw(new_r).reshape(B, C, H, W)
